# baseline (speedup 1.0000x reference)
import jax
import jax.numpy as jnp
from jax import lax
from jax.experimental import pallas as pl
from jax.experimental.pallas import tpu as pltpu
from jax.experimental.pallas import tpu_sc as plsc

NC = 2
NS = 16
NW = NC * NS
L = 16

SDS = jax.ShapeDtypeStruct
F32 = jnp.float32
I32 = jnp.int32


def _silu(u):
    return u * jax.nn.sigmoid(u)


BF16 = jnp.bfloat16


def _pre_body(h_ref, w1a_ref, w1b_ref, a_ref, b_ref):
    h = h_ref[...]
    a_ref[...] = jnp.dot(h, w1a_ref[...], preferred_element_type=F32)
    b_ref[...] = jnp.dot(h, w1b_ref[...], preferred_element_type=F32)


def _precompute(h, w1a, w1b):
    N, D = h.shape
    BN = 2000
    return pl.pallas_call(
        _pre_body,
        grid=(N // BN,),
        in_specs=[
            pl.BlockSpec((BN, D), lambda i: (i, 0)),
            pl.BlockSpec((D, D), lambda i: (0, 0)),
            pl.BlockSpec((D, D), lambda i: (0, 0)),
        ],
        out_specs=[
            pl.BlockSpec((BN, D), lambda i: (i, 0)),
            pl.BlockSpec((BN, D), lambda i: (i, 0)),
        ],
        out_shape=[SDS((N, D), F32), SDS((N, D), F32)],
    )(h, w1a, w1b)


def _sc_gather(row, col, A, B, xq):
    N, D = A.shape
    NX = xq.shape[0]
    E = row.shape[0]
    EPW = E // NW
    K = 80
    full = EPW // K
    G = 8

    mesh = plsc.VectorSubcoreMesh(core_axis_name="c", subcore_axis_name="s")

    def body(row_ref, col_ref, a_ref, b_ref, xq_ref,
             ga_ref, gb_ref, geo_ref,
             idxr_all, idxc_all,
             bufa0, bufb0, gbuf0, bufa1, bufb1, gbuf1,
             bufa2, bufb2, gbuf2,
             xqv, sem_g, sem_w):
        wid = lax.axis_index("s") * NC + lax.axis_index("c")
        wbase = wid * EPW

        pltpu.sync_copy(xq_ref, xqv)
        pltpu.sync_copy(row_ref.at[pl.ds(pl.multiple_of(wbase, 8), EPW)],
                        idxr_all)
        pltpu.sync_copy(col_ref.at[pl.ds(pl.multiple_of(wbase, 8), EPW)],
                        idxc_all)
        lane = lax.iota(I32, L)

        zv = jnp.zeros((L,), F32)
        for i in range(K * G // L):
            gbuf0[pl.ds(i * L, L)] = zv
            gbuf1[pl.ds(i * L, L)] = zv
            gbuf2[pl.ds(i * L, L)] = zv

        def issue_gathers(c, ba, bb):
            cs = pl.multiple_of(c * K, 8)
            pltpu.async_copy(a_ref.at[idxr_all.at[pl.ds(cs, K)]], ba, sem_g)
            pltpu.async_copy(b_ref.at[idxc_all.at[pl.ds(cs, K)]], bb, sem_g)

        def wait_gathers():
            pltpu.make_async_copy(a_ref.at[pl.ds(0, K)], bufa0, sem_g).wait()
            pltpu.make_async_copy(b_ref.at[pl.ds(0, K)], bufb0, sem_g).wait()

        def wait_writes():
            pltpu.make_async_copy(bufa0, ga_ref.at[pl.ds(0, K)], sem_w).wait()
            pltpu.make_async_copy(bufb0, gb_ref.at[pl.ds(0, K)], sem_w).wait()
            pltpu.make_async_copy(gbuf0, geo_ref.at[pl.ds(0, K * G)],
                                  sem_w).wait()

        def geometry(cc, gb):
            cbase = cc * K
            for g in range(K // L):
                iv = cbase + g * L + lane
                i16r = plsc.load_gather(idxr_all, [iv]) * 4
                i16c = plsc.load_gather(idxc_all, [iv]) * 4
                r0 = plsc.load_gather(xqv, [i16r]) - plsc.load_gather(xqv, [i16c])
                r1 = plsc.load_gather(xqv, [i16r + 1]) - plsc.load_gather(xqv, [i16c + 1])
                r2 = plsc.load_gather(xqv, [i16r + 2]) - plsc.load_gather(xqv, [i16c + 2])
                d2v = r0 * r0 + r1 * r1 + r2 * r2
                obase = (g * L + lane) * G
                plsc.store_scatter(gb, [obase], r0)
                plsc.store_scatter(gb, [obase + 1], r1)
                plsc.store_scatter(gb, [obase + 2], r2)
                plsc.store_scatter(gb, [obase + 3], d2v)
                plsc.store_scatter(gb, [obase + 4], r2)

        def issue_writes(cc, ba, bb, gb):
            base = pl.multiple_of(wbase + cc * K, 8)
            pltpu.async_copy(ba, ga_ref.at[pl.ds(base, K)], sem_w)
            pltpu.async_copy(bb, gb_ref.at[pl.ds(base, K)], sem_w)
            pltpu.async_copy(gb, geo_ref.at[pl.ds(pl.multiple_of(base * G, 8),
                                                  K * G)], sem_w)

        def step(cc, cur, nxt):
            ba, bb, gb = cur

            @pl.when(cc >= 2)
            def _():
                wait_writes()

            @pl.when(cc + 1 < full)
            def _():
                issue_gathers(cc + 1, nxt[0], nxt[1])

            wait_gathers()
            geometry(cc, gb)
            issue_writes(cc, ba, bb, gb)

        set0 = (bufa0, bufb0, gbuf0)
        set1 = (bufa1, bufb1, gbuf1)
        set2 = (bufa2, bufb2, gbuf2)
        issue_gathers(0, bufa0, bufb0)

        def loop_body(cc, carry):
            @pl.when(cc % 3 == 0)
            def _():
                step(cc, set0, set1)

            @pl.when(cc % 3 == 1)
            def _():
                step(cc, set1, set2)

            @pl.when(cc % 3 == 2)
            def _():
                step(cc, set2, set0)
            return carry

        lax.fori_loop(0, full, loop_body, 0)
        wait_writes()
        wait_writes()

    fn = pl.kernel(
        body,
        out_type=[SDS((E, D), A.dtype), SDS((E, D), A.dtype),
                  SDS((E * G,), F32)],
        mesh=mesh,
        scratch_types=[
            pltpu.VMEM((EPW,), I32),
            pltpu.VMEM((EPW,), I32),
            pltpu.VMEM((K, D), A.dtype),
            pltpu.VMEM((K, D), A.dtype),
            pltpu.VMEM((K * G,), F32),
            pltpu.VMEM((K, D), A.dtype),
            pltpu.VMEM((K, D), A.dtype),
            pltpu.VMEM((K * G,), F32),
            pltpu.VMEM((K, D), A.dtype),
            pltpu.VMEM((K, D), A.dtype),
            pltpu.VMEM((K * G,), F32),
            pltpu.VMEM((NX,), F32),
            pltpu.SemaphoreType.DMA,
            pltpu.SemaphoreType.DMA,
        ],
        compiler_params=pltpu.CompilerParams(needs_layout_passes=False),
    )
    return fn(row, col, A, B, xq)


def _edge_body(ga_ref, gb_ref, geo_ref, ea_ref,
               w1eg_ref, bm1_ref, wm2_ref, bm2_ref,
               wcv1_ref, bcv1_ref, wcv2_ref, bcv2_ref, p_ref,
               msg_ref, small_ref):
    geo = geo_ref[...]
    d2 = geo[:, 3:4]
    eag = jnp.concatenate([ea_ref[...], geo[:, 3:5]], axis=1)
    pre = (ga_ref[...].astype(F32) + gb_ref[...].astype(F32)
           + jnp.dot(eag, w1eg_ref[...], preferred_element_type=F32)
           + bm1_ref[...])
    m1 = _silu(pre)
    msg = _silu(jnp.dot(m1, wm2_ref[...], preferred_element_type=F32)
                + bm2_ref[...])
    msg_ref[...] = msg
    cv1 = _silu(jnp.dot(msg, wcv1_ref[...], preferred_element_type=F32)
                + bcv1_ref[...])
    cv = jnp.dot(cv1, wcv2_ref[...], preferred_element_type=F32) \
        + bcv2_ref[...]
    inv = 1.0 / (d2 + 1e-8)
    cw = cv[:, 0:1] * inv
    vw = cv[:, 1:2] * inv
    t = jnp.concatenate([geo * cw, geo * vw], axis=1)
    small_ref[...] = jnp.dot(t, p_ref[...], preferred_element_type=F32)


def _edge_mlp(ga, gb, geo, ea, w1eg, bm1, wm2, bm2,
              wcv1, bcv1, wcv2, bcv2, pmat):
    E, D = ga.shape
    G = geo.shape[1]
    EG = w1eg.shape[0]
    wspec = pl.BlockSpec((D, D), lambda i: (0, 0))
    bspec = pl.BlockSpec((1, D), lambda i: (0, 0))
    BE = next(b for b in range(2048, 992, -8) if E % b == 0)
    return pl.pallas_call(
        _edge_body,
        grid=(E // BE,),
        in_specs=[
            pl.BlockSpec((BE, D), lambda i: (i, 0)),
            pl.BlockSpec((BE, D), lambda i: (i, 0)),
            pl.BlockSpec((BE, G), lambda i: (i, 0)),
            pl.BlockSpec((BE, EG - 2), lambda i: (i, 0)),
            pl.BlockSpec((EG, D), lambda i: (0, 0)),
            bspec,
            wspec,
            bspec,
            pl.BlockSpec((D, 2 * D), lambda i: (0, 0)),
            pl.BlockSpec((1, 2 * D), lambda i: (0, 0)),
            pl.BlockSpec((2 * D, 2), lambda i: (0, 0)),
            pl.BlockSpec((1, 2), lambda i: (0, 0)),
            pl.BlockSpec((2 * G, D), lambda i: (0, 0)),
        ],
        out_specs=[
            pl.BlockSpec((BE, D), lambda i: (i, 0)),
            pl.BlockSpec((BE, D), lambda i: (i, 0)),
        ],
        out_shape=[SDS((E, D), F32), SDS((E, D), F32)],
    )(ga, gb, geo, ea, w1eg, bm1, wm2, bm2, wcv1, bcv1, wcv2, bcv2, pmat)


def _sc_scatter(row, msg, small, zeros):
    E, D = msg.shape
    NP = zeros.shape[0]
    EPT = E // NS
    K = 80
    full = EPT // K
    RPT = NP // NS

    mesh = plsc.VectorSubcoreMesh(core_axis_name="c", subcore_axis_name="s")

    def body(row_ref, msg_ref, small_ref, z_ref,
             pm_ref, ps_ref,
             idx0, vb0, idx1, vb1, idx2, vb2,
             acc, sem_l, sem_s):
        cid = lax.axis_index("c")
        sid = lax.axis_index("s")
        rbase = pl.multiple_of(sid * RPT, 8)

        pltpu.sync_copy(z_ref.at[pl.ds(rbase, RPT)],
                        acc.at[pl.ds(rbase, RPT)])
        plsc.subcore_barrier()

        def run(src_ref, out_ref):
            wbase = sid * EPT

            def issue_loads(c, ib, vb):
                base = pl.multiple_of(wbase + c * K, 8)
                pltpu.async_copy(row_ref.at[pl.ds(base, K)], ib, sem_l)
                pltpu.async_copy(src_ref.at[pl.ds(base, K)], vb, sem_l)

            def wait_loads():
                pltpu.make_async_copy(row_ref.at[pl.ds(0, K)], idx0,
                                      sem_l).wait()
                pltpu.make_async_copy(src_ref.at[pl.ds(0, K)], vb0,
                                      sem_l).wait()

            def wait_scatter():
                pltpu.make_async_copy(z_ref.at[pl.ds(0, K)], vb0,
                                      sem_s).wait()

            def step(cc, ib, vb, ibn, vbn):
                @pl.when(cc >= 2)
                def _():
                    wait_scatter()

                @pl.when(cc + 1 < full)
                def _():
                    issue_loads(cc + 1, ibn, vbn)

                wait_loads()
                pltpu.async_copy(vb, acc.at[ib], sem_s, add=True)

            issue_loads(0, idx0, vb0)

            def loop_body(cc, carry):
                @pl.when(cc % 3 == 0)
                def _():
                    step(cc, idx0, vb0, idx1, vb1)

                @pl.when(cc % 3 == 1)
                def _():
                    step(cc, idx1, vb1, idx2, vb2)

                @pl.when(cc % 3 == 2)
                def _():
                    step(cc, idx2, vb2, idx0, vb0)
                return carry

            lax.fori_loop(0, full, loop_body, 0)
            wait_scatter()
            wait_scatter()
            plsc.subcore_barrier()
            pltpu.sync_copy(acc.at[pl.ds(rbase, RPT)],
                            out_ref.at[pl.ds(rbase, RPT)])

        @pl.when(cid == 0)
        def _():
            run(msg_ref, pm_ref)

        @pl.when(cid == 1)
        def _():
            run(small_ref, ps_ref)

    fn = pl.kernel(
        body,
        out_type=[SDS((NP, D), F32), SDS((NP, D), F32)],
        mesh=mesh,
        scratch_types=[
            pltpu.VMEM((K,), I32),
            pltpu.VMEM((K, D), F32),
            pltpu.VMEM((K,), I32),
            pltpu.VMEM((K, D), F32),
            pltpu.VMEM((K,), I32),
            pltpu.VMEM((K, D), F32),
            pltpu.VMEM_SHARED((NP, D), F32),
            pltpu.SemaphoreType.DMA,
            pltpu.SemaphoreType.DMA,
        ],
        compiler_params=pltpu.CompilerParams(needs_layout_passes=False),
    )
    return fn(row, msg, small, zeros)


def _node_update(h, x, v, pms, pss, wn1h, wn1a, bn1, wn2, bn2):
    N, D = h.shape
    BN = 2000
    k = len(pms)

    def body(h_ref, x_ref, v_ref, *refs):
        pm_refs = refs[:k]
        ps_refs = refs[k:2 * k]
        wn1h_ref, wn1a_ref, bn1_ref, wn2_ref, bn2_ref = refs[2 * k:2 * k + 5]
        h_out, x_out, v_out = refs[2 * k + 5:]
        h_ = h_ref[...]
        aggr = pm_refs[0][...]
        s = ps_refs[0][...]
        for r in pm_refs[1:]:
            aggr = aggr + r[...]
        for r in ps_refs[1:]:
            s = s + r[...]
        t = _silu(jnp.dot(h_, wn1h_ref[...], preferred_element_type=F32)
                  + jnp.dot(aggr, wn1a_ref[...], preferred_element_type=F32)
                  + bn1_ref[...])
        h_out[...] = h_ + jnp.dot(t, wn2_ref[...],
                                  preferred_element_type=F32) + bn2_ref[...]
        x_out[...] = x_ref[...] + s[:, 0:3]
        v_out[...] = v_ref[...] + s[:, 3:6]

    wspec = pl.BlockSpec((D, D), lambda i: (0, 0))
    bspec = pl.BlockSpec((1, D), lambda i: (0, 0))
    nspec = pl.BlockSpec((BN, D), lambda i: (i, 0))
    cspec = pl.BlockSpec((BN, 3), lambda i: (i, 0))
    return pl.pallas_call(
        body,
        grid=(N // BN,),
        in_specs=[nspec, cspec, cspec] + [nspec] * (2 * k)
        + [wspec, wspec, bspec, wspec, bspec],
        out_specs=[nspec, cspec, cspec],
        out_shape=[SDS((N, D), F32), SDS((N, 3), F32), SDS((N, 3), F32)],
    )(h, x, v, *pms, *pss, wn1h, wn1a, bn1, wn2, bn2)


def kernel(h, x, v, edge_index, edge_attr,
           Wm1, bm1, Wm2, bm2,
           Wn1, bn1, Wn2, bn2,
           Wc1, bc1, Wc2, bc2,
           Wv1, bv1, Wv2, bv2):
    N, D = h.shape
    H = Wm2.shape[0]
    E = edge_index.shape[1]
    row = edge_index[0]
    col = edge_index[1]

    xq = jnp.pad(x, ((0, 0), (0, 1))).reshape(-1)

    w1a = Wm1[0:D]
    w1b = Wm1[D:2 * D]
    wgeo = Wm1[2 * D:2 * D + 2]
    w1e = Wm1[2 * D + 2:]
    w1eg = jnp.concatenate([w1e, wgeo], axis=0)
    wcv1 = jnp.concatenate([Wc1, Wv1], axis=1)
    bcv1 = jnp.concatenate([bc1, bv1]).reshape(1, 2 * H)
    wcv2 = jnp.concatenate(
        [jnp.concatenate([Wc2, jnp.zeros_like(Wc2)], axis=1),
         jnp.concatenate([jnp.zeros_like(Wv2), Wv2], axis=1)], axis=0)
    bcv2 = jnp.stack([bc2[0], bv2[0]]).reshape(1, 2)
    pmat = jnp.zeros((16, D), F32)
    pmat = pmat.at[0, 0].set(1.0).at[1, 1].set(1.0).at[2, 2].set(1.0)
    pmat = pmat.at[8, 3].set(1.0).at[9, 4].set(1.0).at[10, 5].set(1.0)

    A, B = _precompute(h, w1a, w1b)

    GRAN = NW * 80
    nch = E // GRAN
    npart = 4
    sizes = []
    for i in range(npart):
        c = nch // npart + (1 if i < nch % npart else 0)
        sizes.append(c * GRAN)
    NP = ((N + 8 * NS - 1) // (8 * NS)) * (8 * NS)
    zeros = jnp.zeros((NP, D), F32)

    def part(rr, cc2, ea):
        ga, gb, geo = _sc_gather(rr, cc2, A, B, xq)
        msg, small = _edge_mlp(
            ga, gb, geo.reshape(-1, 8), ea,
            w1eg, bm1.reshape(1, H), Wm2, bm2.reshape(1, H),
            wcv1, bcv1, wcv2, bcv2, pmat)
        return _sc_scatter(rr, msg, small, zeros)

    pms, pss = [], []
    off = 0
    for sz in sizes:
        pm_i, ps_i = part(row[off:off + sz], col[off:off + sz],
                          edge_attr[off:off + sz])
        pms.append(pm_i)
        pss.append(ps_i)
        off += sz

    h_new, x_new, v_new = _node_update(
        h, x, v, pms, pss,
        Wn1[0:D], Wn1[D:], bn1.reshape(1, H), Wn2, bn2.reshape(1, D))
    return (h_new, x_new, v_new)


# speedup vs baseline: 1.0029x; 1.0029x over previous
import jax
import jax.numpy as jnp
from jax import lax
from jax.experimental import pallas as pl
from jax.experimental.pallas import tpu as pltpu
from jax.experimental.pallas import tpu_sc as plsc

NC = 2
NS = 16
NW = NC * NS
L = 16

SDS = jax.ShapeDtypeStruct
F32 = jnp.float32
I32 = jnp.int32


def _silu(u):
    return u * jax.nn.sigmoid(u)


def _pre_body(h_ref, w1a_ref, w1b_ref, a_ref, b_ref):
    h = h_ref[...]
    a_ref[...] = jnp.dot(h, w1a_ref[...], preferred_element_type=F32)
    b_ref[...] = jnp.dot(h, w1b_ref[...], preferred_element_type=F32)


def _precompute(h, w1a, w1b):
    N, D = h.shape
    BN = 2000
    return pl.pallas_call(
        _pre_body,
        grid=(N // BN,),
        in_specs=[
            pl.BlockSpec((BN, D), lambda i: (i, 0)),
            pl.BlockSpec((D, D), lambda i: (0, 0)),
            pl.BlockSpec((D, D), lambda i: (0, 0)),
        ],
        out_specs=[
            pl.BlockSpec((BN, D), lambda i: (i, 0)),
            pl.BlockSpec((BN, D), lambda i: (i, 0)),
        ],
        out_shape=[SDS((N, D), F32), SDS((N, D), F32)],
    )(h, w1a, w1b)


def _sc_gather(row, col, A, B, xq):
    N, D = A.shape
    NX = xq.shape[0]
    E = row.shape[0]
    EPW = E // NW
    K = 80
    full = EPW // K
    G = 8

    mesh = plsc.VectorSubcoreMesh(core_axis_name="c", subcore_axis_name="s")

    def body(row_ref, col_ref, a_ref, b_ref, xq_ref,
             ga_ref, gb_ref, geo_ref,
             idxr_all, idxc_all,
             bufa0, bufb0, gbuf0, bufa1, bufb1, gbuf1,
             bufa2, bufb2, gbuf2,
             xqv, sem_g, sem_w):
        wid = lax.axis_index("s") * NC + lax.axis_index("c")
        wbase = wid * EPW

        pltpu.sync_copy(xq_ref, xqv)
        pltpu.sync_copy(row_ref.at[pl.ds(pl.multiple_of(wbase, 8), EPW)],
                        idxr_all)
        pltpu.sync_copy(col_ref.at[pl.ds(pl.multiple_of(wbase, 8), EPW)],
                        idxc_all)
        lane = lax.iota(I32, L)

        zv = jnp.zeros((L,), F32)
        for i in range(K * G // L):
            gbuf0[pl.ds(i * L, L)] = zv
            gbuf1[pl.ds(i * L, L)] = zv
            gbuf2[pl.ds(i * L, L)] = zv

        def issue_gathers(c, ba, bb):
            cs = pl.multiple_of(c * K, 8)
            pltpu.async_copy(a_ref.at[idxr_all.at[pl.ds(cs, K)]], ba, sem_g)
            pltpu.async_copy(b_ref.at[idxc_all.at[pl.ds(cs, K)]], bb, sem_g)

        def wait_gathers():
            pltpu.make_async_copy(a_ref.at[pl.ds(0, K)], bufa0, sem_g).wait()
            pltpu.make_async_copy(b_ref.at[pl.ds(0, K)], bufb0, sem_g).wait()

        def wait_writes():
            pltpu.make_async_copy(bufa0, ga_ref.at[pl.ds(0, K)], sem_w).wait()
            pltpu.make_async_copy(bufb0, gb_ref.at[pl.ds(0, K)], sem_w).wait()
            pltpu.make_async_copy(gbuf0, geo_ref.at[pl.ds(0, K * G)],
                                  sem_w).wait()

        def geometry(cc, gb):
            cbase = cc * K
            for g in range(K // L):
                iv = cbase + g * L + lane
                i16r = plsc.load_gather(idxr_all, [iv]) * 4
                i16c = plsc.load_gather(idxc_all, [iv]) * 4
                r0 = plsc.load_gather(xqv, [i16r]) - plsc.load_gather(xqv, [i16c])
                r1 = plsc.load_gather(xqv, [i16r + 1]) - plsc.load_gather(xqv, [i16c + 1])
                r2 = plsc.load_gather(xqv, [i16r + 2]) - plsc.load_gather(xqv, [i16c + 2])
                d2v = r0 * r0 + r1 * r1 + r2 * r2
                obase = (g * L + lane) * G
                plsc.store_scatter(gb, [obase], r0)
                plsc.store_scatter(gb, [obase + 1], r1)
                plsc.store_scatter(gb, [obase + 2], r2)
                plsc.store_scatter(gb, [obase + 3], d2v)
                plsc.store_scatter(gb, [obase + 4], r2)

        def issue_writes(cc, ba, bb, gb):
            base = pl.multiple_of(wbase + cc * K, 8)
            pltpu.async_copy(ba, ga_ref.at[pl.ds(base, K)], sem_w)
            pltpu.async_copy(bb, gb_ref.at[pl.ds(base, K)], sem_w)
            pltpu.async_copy(gb, geo_ref.at[pl.ds(pl.multiple_of(base * G, 8),
                                                  K * G)], sem_w)

        def step(cc, cur, nxt):
            ba, bb, gb = cur

            @pl.when(cc >= 2)
            def _():
                wait_writes()

            @pl.when(cc + 1 < full)
            def _():
                issue_gathers(cc + 1, nxt[0], nxt[1])

            geometry(cc, gb)
            wait_gathers()
            issue_writes(cc, ba, bb, gb)

        set0 = (bufa0, bufb0, gbuf0)
        set1 = (bufa1, bufb1, gbuf1)
        set2 = (bufa2, bufb2, gbuf2)
        issue_gathers(0, bufa0, bufb0)

        def loop_body(cc, carry):
            @pl.when(cc % 3 == 0)
            def _():
                step(cc, set0, set1)

            @pl.when(cc % 3 == 1)
            def _():
                step(cc, set1, set2)

            @pl.when(cc % 3 == 2)
            def _():
                step(cc, set2, set0)
            return carry

        lax.fori_loop(0, full, loop_body, 0)
        wait_writes()
        wait_writes()

    fn = pl.kernel(
        body,
        out_type=[SDS((E, D), A.dtype), SDS((E, D), A.dtype),
                  SDS((E * G,), F32)],
        mesh=mesh,
        scratch_types=[
            pltpu.VMEM((EPW,), I32),
            pltpu.VMEM((EPW,), I32),
            pltpu.VMEM((K, D), A.dtype),
            pltpu.VMEM((K, D), A.dtype),
            pltpu.VMEM((K * G,), F32),
            pltpu.VMEM((K, D), A.dtype),
            pltpu.VMEM((K, D), A.dtype),
            pltpu.VMEM((K * G,), F32),
            pltpu.VMEM((K, D), A.dtype),
            pltpu.VMEM((K, D), A.dtype),
            pltpu.VMEM((K * G,), F32),
            pltpu.VMEM((NX,), F32),
            pltpu.SemaphoreType.DMA,
            pltpu.SemaphoreType.DMA,
        ],
        compiler_params=pltpu.CompilerParams(needs_layout_passes=False),
    )
    return fn(row, col, A, B, xq)


def _edge_body(ga_ref, gb_ref, geo_ref, ea_ref,
               w1eg_ref, bm1_ref, wm2_ref, bm2_ref,
               wcv1_ref, bcv1_ref, wcv2_ref, bcv2_ref, p_ref,
               msg_ref, small_ref):
    geo = geo_ref[...]
    d2 = geo[:, 3:4]
    eag = jnp.concatenate([ea_ref[...], geo[:, 3:5]], axis=1)
    pre = (ga_ref[...] + gb_ref[...]
           + jnp.dot(eag, w1eg_ref[...], preferred_element_type=F32)
           + bm1_ref[...])
    m1 = _silu(pre)
    msg = _silu(jnp.dot(m1, wm2_ref[...], preferred_element_type=F32)
                + bm2_ref[...])
    msg_ref[...] = msg
    cv1 = _silu(jnp.dot(msg, wcv1_ref[...], preferred_element_type=F32)
                + bcv1_ref[...])
    cv = jnp.dot(cv1, wcv2_ref[...], preferred_element_type=F32) \
        + bcv2_ref[...]
    inv = 1.0 / (d2 + 1e-8)
    cw = cv[:, 0:1] * inv
    vw = cv[:, 1:2] * inv
    t = jnp.concatenate([geo * cw, geo * vw], axis=1)
    small_ref[...] = jnp.dot(t, p_ref[...], preferred_element_type=F32)


def _edge_mlp(ga, gb, geo, ea, w1eg, bm1, wm2, bm2,
              wcv1, bcv1, wcv2, bcv2, pmat):
    E, D = ga.shape
    G = geo.shape[1]
    EG = w1eg.shape[0]
    wspec = pl.BlockSpec((D, D), lambda i: (0, 0))
    bspec = pl.BlockSpec((1, D), lambda i: (0, 0))
    BE = next(b for b in range(2048, 992, -8) if E % b == 0)
    return pl.pallas_call(
        _edge_body,
        grid=(E // BE,),
        in_specs=[
            pl.BlockSpec((BE, D), lambda i: (i, 0)),
            pl.BlockSpec((BE, D), lambda i: (i, 0)),
            pl.BlockSpec((BE, G), lambda i: (i, 0)),
            pl.BlockSpec((BE, EG - 2), lambda i: (i, 0)),
            pl.BlockSpec((EG, D), lambda i: (0, 0)),
            bspec,
            wspec,
            bspec,
            pl.BlockSpec((D, 2 * D), lambda i: (0, 0)),
            pl.BlockSpec((1, 2 * D), lambda i: (0, 0)),
            pl.BlockSpec((2 * D, 2), lambda i: (0, 0)),
            pl.BlockSpec((1, 2), lambda i: (0, 0)),
            pl.BlockSpec((2 * G, D), lambda i: (0, 0)),
        ],
        out_specs=[
            pl.BlockSpec((BE, D), lambda i: (i, 0)),
            pl.BlockSpec((BE, D), lambda i: (i, 0)),
        ],
        out_shape=[SDS((E, D), F32), SDS((E, D), F32)],
    )(ga, gb, geo, ea, w1eg, bm1, wm2, bm2, wcv1, bcv1, wcv2, bcv2, pmat)


def _sc_scatter(row, msg, small, zeros):
    E, D = msg.shape
    NP = zeros.shape[0]
    EPT = E // NS
    K = 80
    full = EPT // K
    RPT = NP // NS

    mesh = plsc.VectorSubcoreMesh(core_axis_name="c", subcore_axis_name="s")

    def body(row_ref, msg_ref, small_ref, z_ref,
             pm_ref, ps_ref,
             idx0, vb0, idx1, vb1, idx2, vb2,
             acc, sem_l, sem_s):
        cid = lax.axis_index("c")
        sid = lax.axis_index("s")
        rbase = pl.multiple_of(sid * RPT, 8)

        pltpu.sync_copy(z_ref.at[pl.ds(rbase, RPT)],
                        acc.at[pl.ds(rbase, RPT)])
        plsc.subcore_barrier()

        def run(src_ref, out_ref):
            wbase = sid * EPT

            def issue_loads(c, ib, vb):
                base = pl.multiple_of(wbase + c * K, 8)
                pltpu.async_copy(row_ref.at[pl.ds(base, K)], ib, sem_l)
                pltpu.async_copy(src_ref.at[pl.ds(base, K)], vb, sem_l)

            def wait_loads():
                pltpu.make_async_copy(row_ref.at[pl.ds(0, K)], idx0,
                                      sem_l).wait()
                pltpu.make_async_copy(src_ref.at[pl.ds(0, K)], vb0,
                                      sem_l).wait()

            def wait_scatter():
                pltpu.make_async_copy(z_ref.at[pl.ds(0, K)], vb0,
                                      sem_s).wait()

            def step(cc, ib, vb, ibn, vbn):
                @pl.when(cc >= 2)
                def _():
                    wait_scatter()

                @pl.when(cc + 1 < full)
                def _():
                    issue_loads(cc + 1, ibn, vbn)

                wait_loads()
                pltpu.async_copy(vb, acc.at[ib], sem_s, add=True)

            issue_loads(0, idx0, vb0)

            def loop_body(cc, carry):
                @pl.when(cc % 3 == 0)
                def _():
                    step(cc, idx0, vb0, idx1, vb1)

                @pl.when(cc % 3 == 1)
                def _():
                    step(cc, idx1, vb1, idx2, vb2)

                @pl.when(cc % 3 == 2)
                def _():
                    step(cc, idx2, vb2, idx0, vb0)
                return carry

            lax.fori_loop(0, full, loop_body, 0)
            wait_scatter()
            wait_scatter()
            plsc.subcore_barrier()
            pltpu.sync_copy(acc.at[pl.ds(rbase, RPT)],
                            out_ref.at[pl.ds(rbase, RPT)])

        @pl.when(cid == 0)
        def _():
            run(msg_ref, pm_ref)

        @pl.when(cid == 1)
        def _():
            run(small_ref, ps_ref)

    fn = pl.kernel(
        body,
        out_type=[SDS((NP, D), F32), SDS((NP, D), F32)],
        mesh=mesh,
        scratch_types=[
            pltpu.VMEM((K,), I32),
            pltpu.VMEM((K, D), F32),
            pltpu.VMEM((K,), I32),
            pltpu.VMEM((K, D), F32),
            pltpu.VMEM((K,), I32),
            pltpu.VMEM((K, D), F32),
            pltpu.VMEM_SHARED((NP, D), F32),
            pltpu.SemaphoreType.DMA,
            pltpu.SemaphoreType.DMA,
        ],
        compiler_params=pltpu.CompilerParams(needs_layout_passes=False),
    )
    return fn(row, msg, small, zeros)


def _node_update(h, x, v, pms, pss, wn1h, wn1a, bn1, wn2, bn2):
    N, D = h.shape
    BN = 2000
    k = len(pms)

    def body(h_ref, x_ref, v_ref, *refs):
        pm_refs = refs[:k]
        ps_refs = refs[k:2 * k]
        wn1h_ref, wn1a_ref, bn1_ref, wn2_ref, bn2_ref = refs[2 * k:2 * k + 5]
        h_out, x_out, v_out = refs[2 * k + 5:]
        h_ = h_ref[...]
        aggr = pm_refs[0][...]
        s = ps_refs[0][...]
        for r in pm_refs[1:]:
            aggr = aggr + r[...]
        for r in ps_refs[1:]:
            s = s + r[...]
        t = _silu(jnp.dot(h_, wn1h_ref[...], preferred_element_type=F32)
                  + jnp.dot(aggr, wn1a_ref[...], preferred_element_type=F32)
                  + bn1_ref[...])
        h_out[...] = h_ + jnp.dot(t, wn2_ref[...],
                                  preferred_element_type=F32) + bn2_ref[...]
        x_out[...] = x_ref[...] + s[:, 0:3]
        v_out[...] = v_ref[...] + s[:, 3:6]

    wspec = pl.BlockSpec((D, D), lambda i: (0, 0))
    bspec = pl.BlockSpec((1, D), lambda i: (0, 0))
    nspec = pl.BlockSpec((BN, D), lambda i: (i, 0))
    cspec = pl.BlockSpec((BN, 3), lambda i: (i, 0))
    return pl.pallas_call(
        body,
        grid=(N // BN,),
        in_specs=[nspec, cspec, cspec] + [nspec] * (2 * k)
        + [wspec, wspec, bspec, wspec, bspec],
        out_specs=[nspec, cspec, cspec],
        out_shape=[SDS((N, D), F32), SDS((N, 3), F32), SDS((N, 3), F32)],
    )(h, x, v, *pms, *pss, wn1h, wn1a, bn1, wn2, bn2)


def kernel(h, x, v, edge_index, edge_attr,
           Wm1, bm1, Wm2, bm2,
           Wn1, bn1, Wn2, bn2,
           Wc1, bc1, Wc2, bc2,
           Wv1, bv1, Wv2, bv2):
    N, D = h.shape
    H = Wm2.shape[0]
    E = edge_index.shape[1]
    row = edge_index[0]
    col = edge_index[1]

    xq = jnp.pad(x, ((0, 0), (0, 1))).reshape(-1)

    w1a = Wm1[0:D]
    w1b = Wm1[D:2 * D]
    wgeo = Wm1[2 * D:2 * D + 2]
    w1e = Wm1[2 * D + 2:]
    w1eg = jnp.concatenate([w1e, wgeo], axis=0)
    wcv1 = jnp.concatenate([Wc1, Wv1], axis=1)
    bcv1 = jnp.concatenate([bc1, bv1]).reshape(1, 2 * H)
    wcv2 = jnp.concatenate(
        [jnp.concatenate([Wc2, jnp.zeros_like(Wc2)], axis=1),
         jnp.concatenate([jnp.zeros_like(Wv2), Wv2], axis=1)], axis=0)
    bcv2 = jnp.stack([bc2[0], bv2[0]]).reshape(1, 2)
    pmat = jnp.zeros((16, D), F32)
    pmat = pmat.at[0, 0].set(1.0).at[1, 1].set(1.0).at[2, 2].set(1.0)
    pmat = pmat.at[8, 3].set(1.0).at[9, 4].set(1.0).at[10, 5].set(1.0)

    A, B = _precompute(h, w1a, w1b)

    GRAN = NW * 80
    nch = E // GRAN
    npart = 4
    sizes = []
    for i in range(npart):
        c = nch // npart + (1 if i < nch % npart else 0)
        sizes.append(c * GRAN)
    NP = ((N + 8 * NS - 1) // (8 * NS)) * (8 * NS)
    zeros = jnp.zeros((NP, D), F32)

    def part(rr, cc2, ea):
        ga, gb, geo = _sc_gather(rr, cc2, A, B, xq)
        msg, small = _edge_mlp(
            ga, gb, geo.reshape(-1, 8), ea,
            w1eg, bm1.reshape(1, H), Wm2, bm2.reshape(1, H),
            wcv1, bcv1, wcv2, bcv2, pmat)
        return _sc_scatter(rr, msg, small, zeros)

    pms, pss = [], []
    off = 0
    for sz in sizes:
        pm_i, ps_i = part(row[off:off + sz], col[off:off + sz],
                          edge_attr[off:off + sz])
        pms.append(pm_i)
        pss.append(ps_i)
        off += sz

    h_new, x_new, v_new = _node_update(
        h, x, v, pms, pss,
        Wn1[0:D], Wn1[D:], bn1.reshape(1, H), Wn2, bn2.reshape(1, D))
    return (h_new, x_new, v_new)
